# baseline (speedup 1.0000x reference)
import functools
import jax
import jax.numpy as jnp
from jax import lax
from jax.experimental import pallas as pl
from jax.experimental.pallas import tpu as pltpu
from jax.experimental.pallas import tpu_sc as plsc


def kernel(x, emb_table):
    seq, batch, d = x.shape
    d_emb = emb_table.shape[1]
    d_out = d + d_emb
    x2d = x.reshape(seq, batch * d)

    nc, ns = 2, 16
    nw = nc * ns
    rows_per_w = seq // nw
    r = 16
    chunks = rows_per_w // r

    mesh = plsc.VectorSubcoreMesh(
        core_axis_name="c", subcore_axis_name="s", num_cores=nc, num_subcores=ns
    )

    @functools.partial(
        pl.kernel,
        out_type=jax.ShapeDtypeStruct((seq, batch * d_out), jnp.float32),
        mesh=mesh,
        scratch_types=[
            pltpu.VMEM((r, batch * d), jnp.float32),
            pltpu.VMEM((r, d_emb), jnp.float32),
        ],
    )
    def sc_k(x_hbm, emb_hbm, out_hbm, xbuf, ebuf):
        wid = lax.axis_index("s") * nc + lax.axis_index("c")
        base = wid * rows_per_w

        def chunk_body(ci, carry):
            r0 = base + ci * r
            pltpu.sync_copy(x_hbm.at[pl.ds(r0, r), :], xbuf)
            pltpu.sync_copy(emb_hbm.at[pl.ds(r0, r), :], ebuf)
            for b in range(batch):
                pltpu.sync_copy(
                    xbuf.at[:, pl.ds(b * d, d)],
                    out_hbm.at[pl.ds(r0, r), pl.ds(b * d_out, d)],
                )
                pltpu.sync_copy(
                    ebuf,
                    out_hbm.at[pl.ds(r0, r), pl.ds(b * d_out + d, d_emb)],
                )
            return carry

        lax.fori_loop(0, chunks, chunk_body, 0)

    out2d = sc_k(x2d, emb_table)
    return out2d.reshape(seq, batch, d_out)


# speedup vs baseline: 1.0744x; 1.0744x over previous
import jax
import jax.numpy as jnp
from jax.experimental import pallas as pl


def _concat_body(x_ref, emb_ref, out_ref, *, batch, d, d_emb):
    d_out = d + d_emb
    emb = emb_ref[...]
    for b in range(batch):
        out_ref[:, b * d_out:b * d_out + d] = x_ref[:, b * d:(b + 1) * d]
        out_ref[:, b * d_out + d:(b + 1) * d_out] = emb


def kernel(x, emb_table):
    import functools
    seq, batch, d = x.shape
    d_emb = emb_table.shape[1]
    d_out = d + d_emb
    x2d = x.reshape(seq, batch * d)
    bs = 512
    grid = (seq // bs,)
    out2d = pl.pallas_call(
        functools.partial(_concat_body, batch=batch, d=d, d_emb=d_emb),
        grid=grid,
        in_specs=[
            pl.BlockSpec((bs, batch * d), lambda i: (i, 0)),
            pl.BlockSpec((bs, d_emb), lambda i: (i, 0)),
        ],
        out_specs=pl.BlockSpec((bs, batch * d_out), lambda i: (i, 0)),
        out_shape=jax.ShapeDtypeStruct((seq, batch * d_out), x.dtype),
    )(x2d, emb_table)
    return out2d.reshape(seq, batch, d_out)


# speedup vs baseline: 3.4247x; 3.1876x over previous
import functools
import jax
import jax.numpy as jnp
from jax import lax
from jax.experimental import pallas as pl
from jax.experimental.pallas import tpu as pltpu
from jax.experimental.pallas import tpu_sc as plsc


def _x_body(out_alias_ref, x_ref, out_ref):
    out_ref[...] = x_ref[...]


def kernel(x, emb_table):
    seq, batch, d = x.shape
    d_emb = emb_table.shape[1]
    d_out = d + d_emb

    nc, ns = 2, 16
    nw = nc * ns
    rows_per_w = seq // nw
    r = 32
    chunks = rows_per_w // r

    mesh = plsc.VectorSubcoreMesh(
        core_axis_name="c", subcore_axis_name="s", num_cores=nc, num_subcores=ns
    )

    @functools.partial(
        pl.kernel,
        out_type=jax.ShapeDtypeStruct((seq, batch, d_out), jnp.float32),
        mesh=mesh,
        scratch_types=[pltpu.VMEM((r, d_emb), jnp.float32)],
    )
    def sc_fill(emb_hbm, out_hbm, ebuf):
        wid = lax.axis_index("s") * nc + lax.axis_index("c")
        base = wid * rows_per_w
        for ci in range(chunks):
            r0 = base + ci * r
            pltpu.sync_copy(emb_hbm.at[pl.ds(r0, r), :], ebuf)
            for b in range(batch):
                pltpu.sync_copy(ebuf, out_hbm.at[pl.ds(r0, r), b, pl.ds(d, d_emb)])

    out1 = sc_fill(emb_table)

    bs = 512
    grid = (seq // bs,)
    return pl.pallas_call(
        _x_body,
        grid=grid,
        in_specs=[
            pl.BlockSpec(memory_space=pl.ANY),
            pl.BlockSpec((bs, batch, d), lambda i: (i, 0, 0)),
        ],
        out_specs=pl.BlockSpec((bs, batch, d), lambda i: (i, 0, 0)),
        out_shape=jax.ShapeDtypeStruct((seq, batch, d_out), x.dtype),
        input_output_aliases={0: 0},
    )(out1, x)


# speedup vs baseline: 3.4866x; 1.0181x over previous
import functools
import jax
import jax.numpy as jnp
from jax import lax
from jax.experimental import pallas as pl
from jax.experimental.pallas import tpu as pltpu
from jax.experimental.pallas import tpu_sc as plsc


def _x_body(out_alias_ref, x_ref, out_ref):
    out_ref[...] = x_ref[...]


def kernel(x, emb_table):
    seq, batch, d = x.shape
    d_emb = emb_table.shape[1]
    d_out = d + d_emb

    nc, ns = 2, 16
    nw = nc * ns
    rows_per_w = seq // nw
    r = 32
    chunks = rows_per_w // r

    mesh = plsc.VectorSubcoreMesh(
        core_axis_name="c", subcore_axis_name="s", num_cores=nc, num_subcores=ns
    )

    @functools.partial(
        pl.kernel,
        out_type=jax.ShapeDtypeStruct((seq, batch, d_out), jnp.float32),
        mesh=mesh,
        scratch_types=[
            pltpu.VMEM((r, d_emb), jnp.float32),
            pltpu.VMEM((r, d_emb), jnp.float32),
            pltpu.SemaphoreType.DMA,
            pltpu.SemaphoreType.DMA,
            pltpu.SemaphoreType.DMA,
        ],
    )
    def sc_fill(emb_hbm, out_hbm, eb0, eb1, rsem0, rsem1, wsem):
        wid = lax.axis_index("s") * nc + lax.axis_index("c")
        base = wid * rows_per_w
        bufs = (eb0, eb1)
        rsems = (rsem0, rsem1)

        reads = [None] * chunks
        writes = [None] * chunks
        reads[0] = pltpu.async_copy(
            emb_hbm.at[pl.ds(base, r), :], bufs[0], rsems[0]
        )
        for ci in range(chunks):
            if ci >= 1:
                for h in writes[ci - 1]:
                    h.wait()
            if ci + 1 < chunks:
                reads[ci + 1] = pltpu.async_copy(
                    emb_hbm.at[pl.ds(base + (ci + 1) * r, r), :],
                    bufs[(ci + 1) % 2],
                    rsems[(ci + 1) % 2],
                )
            reads[ci].wait()
            r0 = base + ci * r
            writes[ci] = [
                pltpu.async_copy(
                    bufs[ci % 2],
                    out_hbm.at[pl.ds(r0, r), b, pl.ds(d, d_emb)],
                    wsem,
                )
                for b in range(batch)
            ]
        for h in writes[chunks - 1]:
            h.wait()

    out1 = sc_fill(emb_table)

    bs = 512
    grid = (seq // bs,)
    return pl.pallas_call(
        _x_body,
        grid=grid,
        in_specs=[
            pl.BlockSpec(memory_space=pl.ANY),
            pl.BlockSpec((bs, batch, d), lambda i: (i, 0, 0)),
        ],
        out_specs=pl.BlockSpec((bs, batch, d), lambda i: (i, 0, 0)),
        out_shape=jax.ShapeDtypeStruct((seq, batch, d_out), x.dtype),
        input_output_aliases={0: 0},
    )(out1, x)
